# Initial kernel scaffold; baseline (speedup 1.0000x reference)
import jax
import jax.numpy as jnp
from jax.experimental import pallas as pl


def kernel(input, rois):
    raise NotImplementedError("write your pallas kernel here")


# speedup vs baseline: 580.0619x; 580.0619x over previous
import dataclasses

import jax
import jax.numpy as jnp
from jax import lax
from jax.experimental import pallas as pl
from jax.experimental.pallas import tpu as pltpu
from jax.experimental.pallas import tpu_sc as plsc

P = 7
H = W = 100
C_OUT = 8
NB = P * P
K = 5000
NW = 32
PER_W = 157
K_PAD = NW * PER_W
NG = 28
NIDX = NG * 16
NCHUNK = 4
CHUNK = NIDX // NCHUNK
OUT_STRIDE = C_OUT * NB
SCALE = 100.0
ROIS_VLEN = PER_W * 8 + 16


def _full(v):
    return jnp.full((16,), v, jnp.int32)


def _interp_x(xs):
    valid = (xs >= -1.0) & (xs <= float(W))
    c0 = jnp.maximum(xs, 0.0)
    low = c0.astype(jnp.int32)
    lowf = low.astype(jnp.float32)
    cond = low >= W - 1
    frac = jnp.where(cond, 0.0, c0 - lowf)
    vf = jnp.where(valid, 1.0, 0.0)
    w0 = jnp.where(cond, 0.0, 1.0 - frac) * vf
    w1 = jnp.where(cond, 1.0, frac) * vf
    xb = jnp.minimum(low, W - 2)
    return xb, w0, w1


def _interp_y(ys):
    valid = (ys >= -1.0) & (ys <= float(H))
    c0 = jnp.maximum(ys, 0.0)
    low = c0.astype(jnp.int32)
    lowf = low.astype(jnp.float32)
    cond = low >= H - 1
    frac = jnp.where(cond, 0.0, c0 - lowf)
    vf = jnp.where(valid, 1.0, 0.0)
    w0 = (1.0 - frac) * vf
    w1 = frac * vf
    yl = jnp.minimum(low, H - 1)
    yh = jnp.minimum(low + 1, H - 1)
    return yl, yh, w0, w1


def _psroi_body(table_hbm, rois_hbm, out_hbm,
                rois_v, coef_v, coefi_v, idx0, idx1, rows0, rows1,
                outb0, outb1, semg0, semg1, semo0, semo1, semr):
    cid = lax.axis_index("c")
    sid = lax.axis_index("s")
    wid = sid * 2 + cid
    base = wid * PER_W

    pltpu.async_copy(
        rois_hbm.at[pl.ds(pl.multiple_of(base * 8, 8), ROIS_VLEN)],
        rois_v, semr).wait()

    lane = lax.iota(jnp.int32, 16)
    mask8 = lane < 8
    cx = (lane >> 1).astype(jnp.float32) + (lane & 1).astype(jnp.float32) * 0.5 + 0.25
    pwc = jnp.minimum(lane >> 1, P - 1) * (H * W)
    outc = jnp.minimum(lane, C_OUT - 1) * NB

    def axis_setup(li):
        rv = rois_v[pl.ds(pl.multiple_of(li * 8, 8), 16)]
        rs = rv * SCALE - 0.5
        coef_v[pl.ds(0, 16)] = rs
        x1 = plsc.load_gather(coef_v, [_full(1)])
        y1 = plsc.load_gather(coef_v, [_full(2)])
        x2 = plsc.load_gather(coef_v, [_full(3)])
        y2 = plsc.load_gather(coef_v, [_full(4)])
        bw = (x2 - x1) * (1.0 / P)
        bh = (y2 - y1) * (1.0 / P)
        xs = x1 + bw * cx
        ys = y1 + bh * cx
        return xs, ys

    def stage_a(li, idxb, rowsb, semg):
        xs, ys = axis_setup(li)
        xb, _, _ = _interp_x(xs)
        yl, yh, _, _ = _interp_y(ys)
        coefi_v[pl.ds(16, 16)] = yl * W
        coefi_v[pl.ds(32, 16)] = yh * W
        xoff = xb + pwc
        for ph in range(P):
            xoff_ph = xoff + ph * (P * H * W)
            for sy in range(2):
                m = ph * 2 + sy
                ylv = plsc.load_gather(coefi_v, [_full(16 + m)])
                yhv = plsc.load_gather(coefi_v, [_full(32 + m)])
                for tap, yv in enumerate((ylv, yhv)):
                    g = (ph * 2 + sy) * 2 + tap
                    iv = yv + xoff_ph
                    idxb[g // 7, pl.ds((g % 7) * 16, 16)] = iv
        for c in range(NCHUNK):
            pltpu.async_copy(table_hbm.at[idxb.at[c]],
                             rowsb.at[pl.ds(c * CHUNK, CHUNK)], semg)

    def stage_b(li, idxb, rowsb, outb, semg, semo):
        for c in range(NCHUNK):
            pltpu.make_async_copy(table_hbm.at[idxb.at[c]],
                                  rowsb.at[pl.ds(c * CHUNK, CHUNK)],
                                  semg).wait()
        xs, ys = axis_setup(li)
        _, wx0, wx1 = _interp_x(xs)
        _, _, wy0, wy1 = _interp_y(ys)
        coef_v[pl.ds(48, 16)] = wx0
        coef_v[pl.ds(64, 16)] = wx1
        coef_v[pl.ds(80, 16)] = wy0
        coef_v[pl.ds(96, 16)] = wy1
        wx16 = []
        for j in range(14):
            a = plsc.load_gather(coef_v, [_full(48 + j)])
            b = plsc.load_gather(coef_v, [_full(64 + j)])
            wx16.append(jnp.where(mask8, a, b))

        @pl.when(li >= 2)
        def _():
            kprev = base + li - 2
            pltpu.make_async_copy(
                outb.at[pl.ds(0, OUT_STRIDE)],
                out_hbm.at[pl.ds(pl.multiple_of(kprev * OUT_STRIDE, 8),
                                 OUT_STRIDE)],
                semo).wait()

        for ph in range(P):
            wyv = []
            for sy in range(2):
                m = ph * 2 + sy
                wyv.append((plsc.load_gather(coef_v, [_full(80 + m)]),
                            plsc.load_gather(coef_v, [_full(96 + m)])))
            for pw in range(P):
                b = ph * P + pw
                acc = None
                for sy in range(2):
                    w0v, w1v = wyv[sy]
                    gl = (ph * 2 + sy) * 2
                    for sx in range(2):
                        j = pw * 2 + sx
                        rl = rowsb[gl * 16 + j, :]
                        rh = rowsb[(gl + 1) * 16 + j, :]
                        t = rl * w0v + rh * w1v
                        contrib = t * wx16[j]
                        acc = contrib if acc is None else acc + contrib
                s = (acc + lax.rev(acc, (0,))) * 0.25
                plsc.store_scatter(outb, [outc + b], s, mask=mask8)
        kk = base + li
        pltpu.async_copy(
            outb.at[pl.ds(0, OUT_STRIDE)],
            out_hbm.at[pl.ds(pl.multiple_of(kk * OUT_STRIDE, 8), OUT_STRIDE)],
            semo)

    stage_a(0, idx0, rows0, semg0)

    @pl.loop(0, PER_W // 2)
    def _(t):
        li0 = t * 2
        stage_a(li0 + 1, idx1, rows1, semg1)
        stage_b(li0, idx0, rows0, outb0, semg0, semo0)
        stage_a(li0 + 2, idx0, rows0, semg0)
        stage_b(li0 + 1, idx1, rows1, outb1, semg1, semo1)

    stage_b(PER_W - 1, idx0, rows0, outb0, semg0, semo0)
    pltpu.make_async_copy(
        outb1.at[pl.ds(0, OUT_STRIDE)],
        out_hbm.at[pl.ds(pl.multiple_of((base + PER_W - 2) * OUT_STRIDE, 8),
                         OUT_STRIDE)],
        semo1).wait()
    pltpu.make_async_copy(
        outb0.at[pl.ds(0, OUT_STRIDE)],
        out_hbm.at[pl.ds(pl.multiple_of((base + PER_W - 1) * OUT_STRIDE, 8),
                         OUT_STRIDE)],
        semo0).wait()


@jax.jit
def kernel(input, rois):
    x4 = input.reshape(C_OUT, NB, H, W)
    a = jnp.transpose(x4, (1, 2, 3, 0))
    a2 = jnp.concatenate([a[:, :, 1:, :], a[:, :, -1:, :]], axis=2)
    table = jnp.concatenate([a, a2[:, :, :, ::-1]], axis=3).reshape(-1, 16)
    rois8 = jnp.zeros((K_PAD + 16, 8), jnp.float32).at[:K, :5].set(rois)
    rois_flat = rois8.reshape(-1)

    mesh = plsc.VectorSubcoreMesh(core_axis_name="c", subcore_axis_name="s")
    cp = pltpu.CompilerParams()
    if "needs_layout_passes" in pltpu.CompilerParams.__dataclass_fields__:
        cp = dataclasses.replace(cp, needs_layout_passes=False)
    if "use_tc_tiling_on_sc" in pltpu.CompilerParams.__dataclass_fields__:
        cp = dataclasses.replace(cp, use_tc_tiling_on_sc=False)
    run = pl.kernel(
        _psroi_body,
        compiler_params=cp,
        out_type=jax.ShapeDtypeStruct((K_PAD * OUT_STRIDE,), jnp.float32),
        mesh=mesh,
        scratch_types=[
            pltpu.VMEM((ROIS_VLEN,), jnp.float32),
            pltpu.VMEM((128,), jnp.float32),
            pltpu.VMEM((48,), jnp.int32),
            pltpu.VMEM((NCHUNK, CHUNK), jnp.int32),
            pltpu.VMEM((NCHUNK, CHUNK), jnp.int32),
            pltpu.VMEM((NIDX, 16), jnp.float32),
            pltpu.VMEM((NIDX, 16), jnp.float32),
            pltpu.VMEM((400,), jnp.float32),
            pltpu.VMEM((400,), jnp.float32),
            pltpu.SemaphoreType.DMA,
            pltpu.SemaphoreType.DMA,
            pltpu.SemaphoreType.DMA,
            pltpu.SemaphoreType.DMA,
            pltpu.SemaphoreType.DMA,
        ],
    )
    out_flat = run(table, rois_flat)
    return out_flat.reshape(K_PAD, C_OUT, P, P)[:K]
